# baseline (speedup 1.0000x reference)
import functools

import jax
import jax.numpy as jnp
from jax import lax
from jax.experimental import pallas as pl
from jax.experimental.pallas import tpu as pltpu
from jax.experimental.pallas import tpu_sc as plsc


def _gather_kernel(n_rows, embed_dim, n_workers, chunk, nbuf):
    per_w = n_rows // n_workers
    n_chunks = per_w // chunk
    n_outer = n_chunks // nbuf
    mesh = plsc.VectorSubcoreMesh(core_axis_name="c", subcore_axis_name="s")

    @functools.partial(
        pl.kernel,
        out_type=jax.ShapeDtypeStruct((n_rows, embed_dim), jnp.float32),
        mesh=mesh,
        scratch_types=[
            pltpu.VMEM((per_w,), jnp.int32),
            pltpu.VMEM((nbuf, chunk, embed_dim), jnp.float32),
            [pltpu.SemaphoreType.DMA] * nbuf,
            [pltpu.SemaphoreType.DMA] * nbuf,
        ],
        compiler_params=pltpu.CompilerParams(use_tc_tiling_on_sc=False),
    )
    def k(idx_hbm, table_hbm, out_hbm, idx_v, rows_v, gsems, osems):
        wid = lax.axis_index("s") * 2 + lax.axis_index("c")
        base = pl.multiple_of(wid * per_w, chunk)
        pltpu.sync_copy(idx_hbm.at[pl.ds(base, per_w)], idx_v)

        def gather(ci, b):
            return pltpu.make_async_copy(
                table_hbm.at[idx_v.at[pl.ds(ci * chunk, chunk)]],
                rows_v.at[b],
                gsems[b],
            )

        def writeout(ci, b):
            return pltpu.make_async_copy(
                rows_v.at[b],
                out_hbm.at[pl.ds(base + ci * chunk, chunk)],
                osems[b],
            )

        for b in range(nbuf):
            gather(b, b).start()

        def outer(g, carry):
            for b in range(nbuf):
                ci = g * nbuf + b
                gather(ci, b).wait()
                writeout(ci, b).start()
            for b in range(nbuf):
                ci = g * nbuf + b
                writeout(ci, b).wait()
                gather(ci + nbuf, b).start()
            return carry

        lax.fori_loop(0, n_outer - 1, outer, 0)

        last = (n_outer - 1) * nbuf
        for b in range(nbuf):
            gather(last + b, b).wait()
            writeout(last + b, b).start()
        for b in range(nbuf):
            writeout(last + b, b).wait()

    return k


def kernel(inputs, table):
    batch, hist = inputs.shape
    _, embed_dim = table.shape
    n_rows = batch * hist
    idx = inputs.reshape(n_rows).astype(jnp.int32)
    k = _gather_kernel(n_rows, embed_dim, n_workers=32, chunk=1280, nbuf=2)
    out = k(idx, table)
    return out.reshape(batch, hist, embed_dim)


# speedup vs baseline: 1.0024x; 1.0024x over previous
import functools

import jax
import jax.numpy as jnp
from jax import lax
from jax.experimental import pallas as pl
from jax.experimental.pallas import tpu as pltpu
from jax.experimental.pallas import tpu_sc as plsc


def _gather_kernel(n_rows, embed_dim, n_workers, chunk, nbuf):
    per_w = n_rows // n_workers
    n_chunks = per_w // chunk
    n_outer = n_chunks // nbuf
    mesh = plsc.VectorSubcoreMesh(core_axis_name="c", subcore_axis_name="s")

    @functools.partial(
        pl.kernel,
        out_type=jax.ShapeDtypeStruct((n_rows, embed_dim), jnp.float32),
        mesh=mesh,
        scratch_types=[
            pltpu.VMEM((per_w,), jnp.int32),
            pltpu.VMEM((nbuf, chunk, embed_dim), jnp.float32),
            [pltpu.SemaphoreType.DMA] * nbuf,
            [pltpu.SemaphoreType.DMA] * nbuf,
        ],
        compiler_params=pltpu.CompilerParams(use_tc_tiling_on_sc=False),
    )
    def k(idx_hbm, table_hbm, out_hbm, idx_v, rows_v, gsems, osems):
        wid = lax.axis_index("s") * 2 + lax.axis_index("c")
        base = pl.multiple_of(wid * per_w, chunk)
        pltpu.sync_copy(idx_hbm.at[pl.ds(base, per_w)], idx_v)

        def gather(ci, b):
            return pltpu.make_async_copy(
                table_hbm.at[idx_v.at[pl.ds(ci * chunk, chunk)]],
                rows_v.at[b],
                gsems[b],
            )

        def writeout(ci, b):
            return pltpu.make_async_copy(
                rows_v.at[b],
                out_hbm.at[pl.ds(base + ci * chunk, chunk)],
                osems[b],
            )

        for b in range(nbuf):
            gather(b, b).start()

        def outer(g, carry):
            for b in range(nbuf):
                ci = g * nbuf + b
                gather(ci, b).wait()
                writeout(ci, b).start()
            for b in range(nbuf):
                ci = g * nbuf + b
                writeout(ci, b).wait()
                gather(ci + nbuf, b).start()
            return carry

        lax.fori_loop(0, n_outer - 1, outer, 0)

        last = (n_outer - 1) * nbuf
        for b in range(nbuf):
            gather(last + b, b).wait()
            writeout(last + b, b).start()
        for b in range(nbuf):
            writeout(last + b, b).wait()

    return k


def kernel(inputs, table):
    batch, hist = inputs.shape
    _, embed_dim = table.shape
    n_rows = batch * hist
    idx = inputs.reshape(n_rows).astype(jnp.int32)
    k = _gather_kernel(n_rows, embed_dim, n_workers=32, chunk=320, nbuf=10)
    out = k(idx, table)
    return out.reshape(batch, hist, embed_dim)
